# Initial kernel scaffold; baseline (speedup 1.0000x reference)
import jax
import jax.numpy as jnp
from jax.experimental import pallas as pl


def kernel(x, edge_index, W1, b1, W2, b2, W3, b3):
    raise NotImplementedError("write your pallas kernel here")


# speedup vs baseline: 18.6014x; 18.6014x over previous
import jax
import jax.numpy as jnp
from jax import lax
from jax.experimental import pallas as pl
from jax.experimental.pallas import tpu as pltpu
from jax.experimental.pallas import tpu_sc as plsc

N = 10000
E = 320000
CHUNK = 128
NCHUNKS = E // CHUNK
NC = 2
NS = 16
NW = NC * NS
RPT = 640
NPAD = NS * RPT
MAXC = -(-NCHUNKS // NW)

_mesh = plsc.VectorSubcoreMesh(core_axis_name="c", subcore_axis_name="s")


def _deg_body(dst_hbm, ones_hbm, zrows_hbm, out_hbm, dsti, ones_v, accum):
    cid = lax.axis_index("c")
    sid = lax.axis_index("s")
    wid = sid * NC + cid
    pltpu.sync_copy(zrows_hbm, accum.at[pl.ds(sid * RPT, RPT)])
    pltpu.sync_copy(ones_hbm, ones_v)
    plsc.subcore_barrier()

    def step(i, carry):
        c = wid + i * NW

        @pl.when(c < NCHUNKS)
        def _():
            base = c * CHUNK
            pltpu.sync_copy(dst_hbm.at[pl.ds(base, CHUNK)], dsti)
            pltpu.sync_copy(ones_v, accum.at[dsti], add=True)

        return carry

    lax.fori_loop(0, MAXC, step, 0)
    plsc.subcore_barrier()
    pltpu.sync_copy(accum.at[pl.ds(sid * RPT, RPT)],
                    out_hbm.at[cid, pl.ds(sid * RPT, RPT)])


_deg_call = pl.kernel(
    _deg_body,
    out_type=jax.ShapeDtypeStruct((NC, NPAD, 16), jnp.float32),
    mesh=_mesh,
    compiler_params=pltpu.CompilerParams(use_tc_tiling_on_sc=False),
    scratch_types=[
        pltpu.VMEM((CHUNK,), jnp.int32),
        pltpu.VMEM((CHUNK, 16), jnp.float32),
        pltpu.VMEM_SHARED((NPAD, 16), jnp.float32),
    ],
)


def _scatter_body(g_hbm, src_hbm, dst_hbm, zrows_hbm, out_hbm,
                  srci, dsti, rows, accum, sem):
    cid = lax.axis_index("c")
    sid = lax.axis_index("s")
    wid = sid * NC + cid
    pltpu.sync_copy(zrows_hbm, accum.at[pl.ds(sid * RPT, RPT)])
    plsc.subcore_barrier()

    def step(i, carry):
        c = wid + i * NW

        @pl.when(c < NCHUNKS)
        def _():
            base = c * CHUNK
            pltpu.sync_copy(src_hbm.at[pl.ds(base, CHUNK)], srci)
            pltpu.sync_copy(dst_hbm.at[pl.ds(base, CHUNK)], dsti)
            pltpu.async_copy(g_hbm.at[srci], rows, sem).wait()
            pltpu.sync_copy(rows, accum.at[dsti], add=True)

        return carry

    lax.fori_loop(0, MAXC, step, 0)
    plsc.subcore_barrier()
    pltpu.sync_copy(accum.at[pl.ds(sid * RPT, RPT)],
                    out_hbm.at[cid, pl.ds(sid * RPT, RPT)])


def _make_scatter(w):
    return pl.kernel(
        _scatter_body,
        out_type=jax.ShapeDtypeStruct((NC, NPAD, w), jnp.float32),
        mesh=_mesh,
        compiler_params=pltpu.CompilerParams(use_tc_tiling_on_sc=False),
        scratch_types=[
            pltpu.VMEM((CHUNK,), jnp.int32),
            pltpu.VMEM((CHUNK,), jnp.int32),
            pltpu.VMEM((CHUNK, w), jnp.float32),
            pltpu.VMEM_SHARED((NPAD, w), jnp.float32),
            pltpu.SemaphoreType.DMA,
        ],
    )


_scatter32 = _make_scatter(32)
_scatter16 = _make_scatter(16)


def _dinv_from(degp_ref):
    deg = 1.0 + degp_ref[0, 0:N, 0:1] + degp_ref[1, 0:N, 0:1]
    return lax.rsqrt(deg)


def _prep_body(degp_ref, x_ref, w_ref, g_ref):
    dinv = _dinv_from(degp_ref)
    h = jnp.dot(x_ref[...], w_ref[...], preferred_element_type=jnp.float32)
    g_ref[...] = h * dinv


_prep = pl.pallas_call(
    _prep_body, out_shape=jax.ShapeDtypeStruct((N, 32), jnp.float32))


def _combine_body(degp_ref, p_ref, g_ref, w_ref, b_ref, o_ref):
    dinv = _dinv_from(degp_ref)
    s = p_ref[0, 0:N, :] + p_ref[1, 0:N, :] + g_ref[...]
    pre = s * dinv + b_ref[...]
    a = jnp.where(pre > 0, pre, jnp.exp(jnp.minimum(pre, 0.0)) - 1.0)
    o_ref[...] = jnp.dot(a, w_ref[...],
                         preferred_element_type=jnp.float32) * dinv


def _make_combine(wout):
    return pl.pallas_call(
        _combine_body, out_shape=jax.ShapeDtypeStruct((N, wout), jnp.float32))


_combine32 = _make_combine(32)
_combine16 = _make_combine(16)


def _final_body(degp_ref, p_ref, g_ref, b_ref, o_ref):
    dinv = _dinv_from(degp_ref)
    o_ref[...] = (p_ref[0, 0:N, :] + p_ref[1, 0:N, :] + g_ref[...]) * dinv + b_ref[...]


_final = pl.pallas_call(
    _final_body, out_shape=jax.ShapeDtypeStruct((N, 16), jnp.float32))


def kernel(x, edge_index, W1, b1, W2, b2, W3, b3):
    ei = edge_index.astype(jnp.int32)
    src, dst = ei[0], ei[1]
    f32 = jnp.float32
    W1p = jnp.zeros((128, 32), f32).at[:, :30].set(W1)
    W2p = jnp.zeros((32, 32), f32).at[:30, :30].set(W2)
    W3p = jnp.zeros((32, 16), f32).at[:30, :10].set(W3)
    b1p = jnp.zeros((1, 32), f32).at[0, :30].set(b1)
    b2p = jnp.zeros((1, 32), f32).at[0, :30].set(b2)
    b3p = jnp.zeros((1, 16), f32).at[0, :10].set(b3)
    ones16 = jnp.ones((CHUNK, 16), f32)
    z16 = jnp.zeros((RPT, 16), f32)
    z32 = jnp.zeros((RPT, 32), f32)

    degp = _deg_call(dst, ones16, z16)
    g1 = _prep(degp, x, W1p)
    p1 = _scatter32(g1, src, dst, z32)
    g2 = _combine32(degp, p1, g1, W2p, b1p)
    p2 = _scatter32(g2, src, dst, z32)
    g3 = _combine16(degp, p2, g2, W3p, b2p)
    p3 = _scatter16(g3, src, dst, z16)
    out = _final(degp, p3, g3, b3p)
    return out[:, :10]


# speedup vs baseline: 49.5203x; 2.6622x over previous
import jax
import jax.numpy as jnp
from jax import lax
from jax.experimental import pallas as pl
from jax.experimental.pallas import tpu as pltpu
from jax.experimental.pallas import tpu_sc as plsc

N = 10000
E = 320000
NC = 2
NS = 16
NW = NC * NS
CHUNK = 1000
NCH = E // (NW * CHUNK)
RPT = 640
NPAD = NS * RPT

_mesh = plsc.VectorSubcoreMesh(core_axis_name="c", subcore_axis_name="s")
_sc_params = pltpu.CompilerParams(use_tc_tiling_on_sc=False)


def _deg_body(dst2_hbm, ones_hbm, zrows_hbm, out_hbm, dsti, ones_v, accum, sem):
    cid = lax.axis_index("c")
    sid = lax.axis_index("s")
    wid = sid * NC + cid
    pltpu.sync_copy(zrows_hbm, accum.at[pl.ds(sid * RPT, RPT)])
    pltpu.sync_copy(dst2_hbm.at[pl.ds(wid * NCH, NCH)], dsti)
    pltpu.sync_copy(ones_hbm, ones_v)
    plsc.subcore_barrier()
    descs = []
    for i in range(NCH):
        descs.append(
            pltpu.async_copy(ones_v, accum.at[dsti.at[i]], sem, add=True))
    for d in descs:
        d.wait()
    plsc.subcore_barrier()
    pltpu.sync_copy(accum.at[pl.ds(sid * RPT, RPT)],
                    out_hbm.at[cid, pl.ds(sid * RPT, RPT)])


_deg_call = pl.kernel(
    _deg_body,
    out_type=jax.ShapeDtypeStruct((NC, NPAD, 16), jnp.float32),
    mesh=_mesh,
    compiler_params=_sc_params,
    scratch_types=[
        pltpu.VMEM((NCH, CHUNK), jnp.int32),
        pltpu.VMEM((CHUNK, 16), jnp.float32),
        pltpu.VMEM_SHARED((NPAD, 16), jnp.float32),
        pltpu.SemaphoreType.DMA,
    ],
)


def _scatter_body(g_hbm, src2_hbm, dst2_hbm, zrows_hbm, out_hbm,
                  srci, dsti, rows0, rows1, accum, sg, ssc0, ssc1):
    cid = lax.axis_index("c")
    sid = lax.axis_index("s")
    wid = sid * NC + cid
    pltpu.sync_copy(zrows_hbm, accum.at[pl.ds(sid * RPT, RPT)])
    pltpu.sync_copy(src2_hbm.at[pl.ds(wid * NCH, NCH)], srci)
    pltpu.sync_copy(dst2_hbm.at[pl.ds(wid * NCH, NCH)], dsti)
    plsc.subcore_barrier()
    rows = (rows0, rows1)
    ssc = (ssc0, ssc1)
    sd = [None, None]
    for i in range(NCH):
        b = i % 2
        if sd[b] is not None:
            sd[b].wait()
        pltpu.async_copy(g_hbm.at[srci.at[i]], rows[b], sg).wait()
        sd[b] = pltpu.async_copy(rows[b], accum.at[dsti.at[i]], ssc[b],
                                 add=True)
    sd[0].wait()
    sd[1].wait()
    plsc.subcore_barrier()
    pltpu.sync_copy(accum.at[pl.ds(sid * RPT, RPT)],
                    out_hbm.at[cid, pl.ds(sid * RPT, RPT)])


def _make_scatter(w):
    return pl.kernel(
        _scatter_body,
        out_type=jax.ShapeDtypeStruct((NC, NPAD, w), jnp.float32),
        mesh=_mesh,
        compiler_params=_sc_params,
        scratch_types=[
            pltpu.VMEM((NCH, CHUNK), jnp.int32),
            pltpu.VMEM((NCH, CHUNK), jnp.int32),
            pltpu.VMEM((CHUNK, w), jnp.float32),
            pltpu.VMEM((CHUNK, w), jnp.float32),
            pltpu.VMEM_SHARED((NPAD, w), jnp.float32),
            pltpu.SemaphoreType.DMA,
            pltpu.SemaphoreType.DMA,
            pltpu.SemaphoreType.DMA,
        ],
    )


_scatter32 = _make_scatter(32)
_scatter16 = _make_scatter(16)


def _dinv_from(degp_ref):
    deg = 1.0 + degp_ref[0, 0:N, 0:1] + degp_ref[1, 0:N, 0:1]
    return lax.rsqrt(deg)


def _prep_body(degp_ref, x_ref, w_ref, g_ref):
    dinv = _dinv_from(degp_ref)
    h = jnp.dot(x_ref[...], w_ref[...], preferred_element_type=jnp.float32)
    g_ref[...] = h * dinv


_prep = pl.pallas_call(
    _prep_body, out_shape=jax.ShapeDtypeStruct((N, 32), jnp.float32))


def _combine_body(degp_ref, p_ref, g_ref, w_ref, b_ref, o_ref):
    dinv = _dinv_from(degp_ref)
    s = p_ref[0, 0:N, :] + p_ref[1, 0:N, :] + g_ref[...]
    pre = s * dinv + b_ref[...]
    a = jnp.where(pre > 0, pre, jnp.exp(jnp.minimum(pre, 0.0)) - 1.0)
    o_ref[...] = jnp.dot(a, w_ref[...],
                         preferred_element_type=jnp.float32) * dinv


def _make_combine(wout):
    return pl.pallas_call(
        _combine_body, out_shape=jax.ShapeDtypeStruct((N, wout), jnp.float32))


_combine32 = _make_combine(32)
_combine16 = _make_combine(16)


def _final_body(degp_ref, p_ref, g_ref, b_ref, o_ref):
    dinv = _dinv_from(degp_ref)
    o_ref[...] = (p_ref[0, 0:N, :] + p_ref[1, 0:N, :] + g_ref[...]) * dinv + b_ref[...]


_final = pl.pallas_call(
    _final_body, out_shape=jax.ShapeDtypeStruct((N, 16), jnp.float32))


def kernel(x, edge_index, W1, b1, W2, b2, W3, b3):
    ei = edge_index.astype(jnp.int32)
    src2 = ei[0].reshape(NW * NCH, CHUNK)
    dst2 = ei[1].reshape(NW * NCH, CHUNK)
    f32 = jnp.float32
    W1p = jnp.zeros((128, 32), f32).at[:, :30].set(W1)
    W2p = jnp.zeros((32, 32), f32).at[:30, :30].set(W2)
    W3p = jnp.zeros((32, 16), f32).at[:30, :10].set(W3)
    b1p = jnp.zeros((1, 32), f32).at[0, :30].set(b1)
    b2p = jnp.zeros((1, 32), f32).at[0, :30].set(b2)
    b3p = jnp.zeros((1, 16), f32).at[0, :10].set(b3)
    ones16 = jnp.ones((CHUNK, 16), f32)
    z16 = jnp.zeros((RPT, 16), f32)
    z32 = jnp.zeros((RPT, 32), f32)

    degp = _deg_call(dst2, ones16, z16)
    g1 = _prep(degp, x, W1p)
    p1 = _scatter32(g1, src2, dst2, z32)
    g2 = _combine32(degp, p1, g1, W2p, b1p)
    p2 = _scatter32(g2, src2, dst2, z32)
    g3 = _combine16(degp, p2, g2, W3p, b2p)
    p3 = _scatter16(g3, src2, dst2, z16)
    out = _final(degp, p3, g3, b3p)
    return out[:, :10]


# speedup vs baseline: 63.5117x; 1.2825x over previous
import jax
import jax.numpy as jnp
from jax import lax
from jax.experimental import pallas as pl
from jax.experimental.pallas import tpu as pltpu
from jax.experimental.pallas import tpu_sc as plsc

N = 10000
E = 320000
NC = 2
NS = 16
NW = NC * NS
CHUNK = 1000
NCH = E // (NW * CHUNK)
RPT = 640
NPAD = NS * RPT
NF = N // 4

_mesh = plsc.VectorSubcoreMesh(core_axis_name="c", subcore_axis_name="s")
_sc_params = pltpu.CompilerParams(use_tc_tiling_on_sc=False)


def _deg_body(dst2_hbm, ones_hbm, zrows_hbm, out_hbm, dsti, ones_v, accum, sem):
    cid = lax.axis_index("c")
    sid = lax.axis_index("s")
    wid = sid * NC + cid
    pltpu.sync_copy(zrows_hbm, accum.at[pl.ds(sid * RPT, RPT)])
    pltpu.sync_copy(dst2_hbm.at[pl.ds(wid * NCH, NCH)], dsti)
    pltpu.sync_copy(ones_hbm, ones_v)
    plsc.subcore_barrier()
    descs = []
    for i in range(NCH):
        descs.append(
            pltpu.async_copy(ones_v, accum.at[dsti.at[i]], sem, add=True))
    for d in descs:
        d.wait()
    plsc.subcore_barrier()
    pltpu.sync_copy(accum.at[pl.ds(sid * RPT, RPT)],
                    out_hbm.at[cid, pl.ds(sid * RPT, RPT)])


_deg_call = pl.kernel(
    _deg_body,
    out_type=jax.ShapeDtypeStruct((NC, NPAD, 32), jnp.float32),
    mesh=_mesh,
    compiler_params=_sc_params,
    scratch_types=[
        pltpu.VMEM((NCH, CHUNK), jnp.int32),
        pltpu.VMEM((CHUNK, 32), jnp.float32),
        pltpu.VMEM_SHARED((NPAD, 32), jnp.float32),
        pltpu.SemaphoreType.DMA,
    ],
)


def _scatter_body(g_hbm, src2_hbm, dst2_hbm, zrows_hbm, out_hbm,
                  srci, dsti, rows0, rows1, accum, sg, ssc0, ssc1):
    cid = lax.axis_index("c")
    sid = lax.axis_index("s")
    wid = sid * NC + cid
    pltpu.sync_copy(zrows_hbm, accum.at[pl.ds(sid * RPT, RPT)])
    pltpu.sync_copy(src2_hbm.at[pl.ds(wid * NCH, NCH)], srci)
    pltpu.sync_copy(dst2_hbm.at[pl.ds(wid * NCH, NCH)], dsti)
    plsc.subcore_barrier()
    rows = (rows0, rows1)
    ssc = (ssc0, ssc1)
    sd = [None, None]
    for i in range(NCH):
        b = i % 2
        if sd[b] is not None:
            sd[b].wait()
        pltpu.async_copy(g_hbm.at[srci.at[i]], rows[b], sg).wait()
        sd[b] = pltpu.async_copy(rows[b], accum.at[dsti.at[i]], ssc[b],
                                 add=True)
    sd[0].wait()
    sd[1].wait()
    plsc.subcore_barrier()
    pltpu.sync_copy(accum.at[pl.ds(sid * RPT, RPT)],
                    out_hbm.at[cid, pl.ds(sid * RPT, RPT)])


_scatter32 = pl.kernel(
    _scatter_body,
    out_type=jax.ShapeDtypeStruct((NC, NPAD, 32), jnp.float32),
    mesh=_mesh,
    compiler_params=_sc_params,
    scratch_types=[
        pltpu.VMEM((NCH, CHUNK), jnp.int32),
        pltpu.VMEM((NCH, CHUNK), jnp.int32),
        pltpu.VMEM((CHUNK, 32), jnp.float32),
        pltpu.VMEM((CHUNK, 32), jnp.float32),
        pltpu.VMEM_SHARED((NPAD, 32), jnp.float32),
        pltpu.SemaphoreType.DMA,
        pltpu.SemaphoreType.DMA,
        pltpu.SemaphoreType.DMA,
    ],
)


def _dinv_from(degp_ref):
    deg = 1.0 + degp_ref[0, 0:NF, :] + degp_ref[1, 0:NF, :]
    return lax.rsqrt(deg)


def _prep_body(degp_ref, xf_ref, w_ref, g_ref):
    dinv = _dinv_from(degp_ref)
    h = jnp.dot(xf_ref[...], w_ref[...], preferred_element_type=jnp.float32)
    g_ref[...] = h * dinv


_prep = pl.pallas_call(
    _prep_body, out_shape=jax.ShapeDtypeStruct((NF, 128), jnp.float32))


def _combine_body(degp_ref, p_ref, g_ref, w_ref, b_ref, o_ref):
    dinv = _dinv_from(degp_ref)
    s = p_ref[0, 0:NF, :] + p_ref[1, 0:NF, :] + g_ref[...]
    pre = s * dinv + b_ref[...]
    a = jnp.where(pre > 0, pre, jnp.exp(jnp.minimum(pre, 0.0)) - 1.0)
    o_ref[...] = jnp.dot(a, w_ref[...],
                         preferred_element_type=jnp.float32) * dinv


_combine = pl.pallas_call(
    _combine_body, out_shape=jax.ShapeDtypeStruct((NF, 128), jnp.float32))


def _final_body(degp_ref, p_ref, g_ref, b_ref, o_ref):
    dinv = _dinv_from(degp_ref)
    o_ref[...] = ((p_ref[0, 0:NF, :] + p_ref[1, 0:NF, :] + g_ref[...])
                  * dinv + b_ref[...])


_final = pl.pallas_call(
    _final_body, out_shape=jax.ShapeDtypeStruct((NF, 128), jnp.float32))


def _blockdiag4(w32):
    z = jnp.zeros((128, 128), jnp.float32)
    for k in range(4):
        z = z.at[32 * k:32 * (k + 1), 32 * k:32 * (k + 1)].set(w32)
    return z


def kernel(x, edge_index, W1, b1, W2, b2, W3, b3):
    ei = edge_index.astype(jnp.int32)
    src2 = ei[0].reshape(NW * NCH, CHUNK)
    dst2 = ei[1].reshape(NW * NCH, CHUNK)
    f32 = jnp.float32

    xf = x.reshape(NF, 512)
    W1p = jnp.zeros((128, 32), f32).at[:, :30].set(W1)
    W1blk = jnp.zeros((512, 128), f32)
    for k in range(4):
        W1blk = W1blk.at[128 * k:128 * (k + 1), 32 * k:32 * (k + 1)].set(W1p)
    W2blk = _blockdiag4(jnp.zeros((32, 32), f32).at[:30, :30].set(W2))
    W3blk = _blockdiag4(jnp.zeros((32, 32), f32).at[:30, :10].set(W3))
    b1t = jnp.tile(jnp.zeros((1, 32), f32).at[0, :30].set(b1), (1, 4))
    b2t = jnp.tile(jnp.zeros((1, 32), f32).at[0, :30].set(b2), (1, 4))
    b3t = jnp.tile(jnp.zeros((1, 32), f32).at[0, :10].set(b3), (1, 4))
    ones32 = jnp.ones((CHUNK, 32), f32)
    z32 = jnp.zeros((RPT, 32), f32)

    def fold(p):
        return p.reshape(NC, NPAD // 4, 128)

    degp = fold(_deg_call(dst2, ones32, z32))
    g1 = _prep(degp, xf, W1blk)
    p1 = fold(_scatter32(g1.reshape(N, 32), src2, dst2, z32))
    g2 = _combine(degp, p1, g1, W2blk, b1t)
    p2 = fold(_scatter32(g2.reshape(N, 32), src2, dst2, z32))
    g3 = _combine(degp, p2, g2, W3blk, b2t)
    p3 = fold(_scatter32(g3.reshape(N, 32), src2, dst2, z32))
    out = _final(degp, p3, g3, b3t)
    return out.reshape(N, 32)[:, :10]


# speedup vs baseline: 70.8742x; 1.1159x over previous
import jax
import jax.numpy as jnp
from jax import lax
from jax.experimental import pallas as pl
from jax.experimental.pallas import tpu as pltpu
from jax.experimental.pallas import tpu_sc as plsc

N = 10000
E = 320000
NC = 2
NS = 16
NW = NC * NS
CHUNK = 400
NCH = E // (NW * CHUNK)
NBUF = 4
TPW = NCH * CHUNK
RPT = 640
NPAD = NS * RPT
NF = N // 4

_mesh = plsc.VectorSubcoreMesh(core_axis_name="c", subcore_axis_name="s")
_sc_params = pltpu.CompilerParams(use_tc_tiling_on_sc=False)


def _deg_body(dst_hbm, ones_hbm, zrows_hbm, out_hbm, dsti, ones_v, accum, sem):
    cid = lax.axis_index("c")
    sid = lax.axis_index("s")
    wid = sid * NC + cid
    pltpu.sync_copy(zrows_hbm, accum.at[pl.ds(sid * RPT, RPT)])
    idx_descs = [
        pltpu.async_copy(dst_hbm.at[pl.ds(wid * TPW + i * CHUNK, CHUNK)],
                         dsti.at[i], sem)
        for i in range(NCH)
    ]
    for d in idx_descs:
        d.wait()
    pltpu.sync_copy(ones_hbm, ones_v)
    plsc.subcore_barrier()
    descs = [
        pltpu.async_copy(ones_v, accum.at[dsti.at[i]], sem, add=True)
        for i in range(NCH)
    ]
    for d in descs:
        d.wait()
    plsc.subcore_barrier()
    pltpu.sync_copy(accum.at[pl.ds(sid * RPT, RPT)],
                    out_hbm.at[cid, pl.ds(sid * RPT, RPT), pl.ds(0, 16)])
    pltpu.sync_copy(accum.at[pl.ds(sid * RPT, RPT)],
                    out_hbm.at[cid, pl.ds(sid * RPT, RPT), pl.ds(16, 16)])


_deg_call = pl.kernel(
    _deg_body,
    out_type=jax.ShapeDtypeStruct((NC, NPAD, 32), jnp.float32),
    mesh=_mesh,
    compiler_params=_sc_params,
    scratch_types=[
        pltpu.VMEM((NCH, CHUNK), jnp.int32),
        pltpu.VMEM((CHUNK, 16), jnp.float32),
        pltpu.VMEM_SHARED((NPAD, 16), jnp.float32),
        pltpu.SemaphoreType.DMA,
    ],
)


def _scatter_body(g_hbm, src_hbm, dst_hbm, zrows_hbm, out_hbm,
                  srci, dsti, rows, accum, sg, ssc, si):
    cid = lax.axis_index("c")
    sid = lax.axis_index("s")
    wid = sid * NC + cid
    pltpu.sync_copy(zrows_hbm, accum.at[pl.ds(sid * RPT, RPT)])
    idx_descs = [pltpu.async_copy(src_hbm.at[pl.ds(wid * TPW, TPW)], srci, si)]
    idx_descs += [
        pltpu.async_copy(dst_hbm.at[pl.ds(wid * TPW + i * CHUNK, CHUNK)],
                         dsti.at[i], si)
        for i in range(NCH)
    ]
    for d in idx_descs:
        d.wait()
    plsc.subcore_barrier()

    def start_gather(i):
        b = i % NBUF
        return pltpu.async_copy(
            g_hbm.at[srci.at[pl.ds(i * CHUNK, CHUNK)]], rows[b], sg[b])

    gd = [None] * NCH
    sd = [None] * NCH
    for j in range(NBUF - 1):
        gd[j] = start_gather(j)
    last_waited = -1
    for i in range(NCH):
        b = i % NBUF
        gd[i].wait()
        sd[i] = pltpu.async_copy(rows[b], accum.at[dsti.at[i]], ssc[b],
                                 add=True)
        j = i + NBUF - 1
        if j < NCH:
            if i >= 1:
                sd[i - 1].wait()
                last_waited = i - 1
            gd[j] = start_gather(j)
    for k in range(last_waited + 1, NCH):
        sd[k].wait()
    plsc.subcore_barrier()
    pltpu.sync_copy(accum.at[pl.ds(sid * RPT, RPT)],
                    out_hbm.at[cid, pl.ds(sid * RPT, RPT)])


_scatter32 = pl.kernel(
    _scatter_body,
    out_type=jax.ShapeDtypeStruct((NC, NPAD, 32), jnp.float32),
    mesh=_mesh,
    compiler_params=_sc_params,
    scratch_types=[
        pltpu.VMEM((TPW,), jnp.int32),
        pltpu.VMEM((NCH, CHUNK), jnp.int32),
        [pltpu.VMEM((CHUNK, 32), jnp.float32) for _ in range(NBUF)],
        pltpu.VMEM_SHARED((NPAD, 32), jnp.float32),
        [pltpu.SemaphoreType.DMA for _ in range(NBUF)],
        [pltpu.SemaphoreType.DMA for _ in range(NBUF)],
        pltpu.SemaphoreType.DMA,
    ],
)


def _split_body(ei_ref, src_ref, dst_ref):
    src_ref[...] = ei_ref[0, :]
    dst_ref[...] = ei_ref[1, :]


_split = pl.pallas_call(
    _split_body,
    out_shape=(jax.ShapeDtypeStruct((E,), jnp.int32),
               jax.ShapeDtypeStruct((E,), jnp.int32)))


def _dinv_from(degp_ref):
    deg = 1.0 + degp_ref[0, 0:NF, :] + degp_ref[1, 0:NF, :]
    return lax.rsqrt(deg)


def _prep_body(degp_ref, xf_ref, w_ref, g_ref):
    dinv = _dinv_from(degp_ref)
    h = jnp.dot(xf_ref[...], w_ref[...], preferred_element_type=jnp.float32)
    g_ref[...] = h * dinv


_prep = pl.pallas_call(
    _prep_body, out_shape=jax.ShapeDtypeStruct((NF, 128), jnp.float32))


def _combine_body(degp_ref, p_ref, g_ref, w_ref, b_ref, o_ref):
    dinv = _dinv_from(degp_ref)
    s = p_ref[0, 0:NF, :] + p_ref[1, 0:NF, :] + g_ref[...]
    pre = s * dinv + b_ref[...]
    a = jnp.where(pre > 0, pre, jnp.exp(jnp.minimum(pre, 0.0)) - 1.0)
    o_ref[...] = jnp.dot(a, w_ref[...],
                         preferred_element_type=jnp.float32) * dinv


_combine = pl.pallas_call(
    _combine_body, out_shape=jax.ShapeDtypeStruct((NF, 128), jnp.float32))


def _final_body(degp_ref, p_ref, g_ref, b_ref, o_ref):
    dinv = _dinv_from(degp_ref)
    o_ref[...] = ((p_ref[0, 0:NF, :] + p_ref[1, 0:NF, :] + g_ref[...])
                  * dinv + b_ref[...])


_final = pl.pallas_call(
    _final_body, out_shape=jax.ShapeDtypeStruct((NF, 128), jnp.float32))


def _blockdiag4(w32):
    z = jnp.zeros((32, 32), jnp.float32)
    cols = [jnp.concatenate([z] * k + [w32] + [z] * (3 - k), axis=0)
            for k in range(4)]
    return jnp.concatenate(cols, axis=1)


def _pad2d(w, rows, cols):
    r, c = w.shape
    w = jnp.concatenate([w, jnp.zeros((rows - r, c), jnp.float32)], axis=0)
    return jnp.concatenate([w, jnp.zeros((rows, cols - c), jnp.float32)],
                           axis=1)


def kernel(x, edge_index, W1, b1, W2, b2, W3, b3):
    ei = edge_index.astype(jnp.int32)
    src, dst = _split(ei)
    f32 = jnp.float32

    xf = x.reshape(NF, 512)
    W1p = _pad2d(W1, 128, 32)
    z1 = jnp.zeros((128, 32), f32)
    W1blk = jnp.concatenate(
        [jnp.concatenate([z1] * k + [W1p] + [z1] * (3 - k), axis=0)
         for k in range(4)], axis=1).reshape(512, 128)
    W2blk = _blockdiag4(_pad2d(W2, 32, 32))
    W3blk = _blockdiag4(_pad2d(W3, 32, 32))

    def tile_bias(b):
        bp = jnp.concatenate([b, jnp.zeros((32 - b.shape[0],), f32)])
        return jnp.tile(bp, 4).reshape(1, 128)

    b1t, b2t, b3t = tile_bias(b1), tile_bias(b2), tile_bias(b3)
    ones16 = jnp.ones((CHUNK, 16), f32)
    z16 = jnp.zeros((RPT, 16), f32)
    z32 = jnp.zeros((RPT, 32), f32)

    def fold(p):
        return p.reshape(NC, NPAD // 4, 128)

    degp = fold(_deg_call(dst, ones16, z16))
    g1 = _prep(degp, xf, W1blk)
    p1 = fold(_scatter32(g1.reshape(N, 32), src, dst, z32))
    g2 = _combine(degp, p1, g1, W2blk, b1t)
    p2 = fold(_scatter32(g2.reshape(N, 32), src, dst, z32))
    g3 = _combine(degp, p2, g2, W3blk, b2t)
    p3 = fold(_scatter32(g3.reshape(N, 32), src, dst, z32))
    out = _final(degp, p3, g3, b3t)
    return out.reshape(N, 32)[:, :10]


# speedup vs baseline: 71.4271x; 1.0078x over previous
import jax
import jax.numpy as jnp
from jax import lax
from jax.experimental import pallas as pl
from jax.experimental.pallas import tpu as pltpu
from jax.experimental.pallas import tpu_sc as plsc

N = 10000
E = 320000
NC = 2
NS = 16
NW = NC * NS
CHUNK = 400
NCH = E // (NW * CHUNK)
NBUF = 6
TPW = NCH * CHUNK
RPT = 640
NPAD = NS * RPT
NF = N // 4

_mesh = plsc.VectorSubcoreMesh(core_axis_name="c", subcore_axis_name="s")
_sc_params = pltpu.CompilerParams(use_tc_tiling_on_sc=False)


def _deg_body(dst_hbm, ones_hbm, zrows_hbm, out_hbm, dsti, ones_v, accum, sem):
    cid = lax.axis_index("c")
    sid = lax.axis_index("s")
    wid = sid * NC + cid
    pltpu.sync_copy(zrows_hbm, accum.at[pl.ds(sid * RPT, RPT)])
    idx_descs = [
        pltpu.async_copy(dst_hbm.at[pl.ds(wid * TPW + i * CHUNK, CHUNK)],
                         dsti.at[i], sem)
        for i in range(NCH)
    ]
    for d in idx_descs:
        d.wait()
    pltpu.sync_copy(ones_hbm, ones_v)
    plsc.subcore_barrier()
    descs = [
        pltpu.async_copy(ones_v, accum.at[dsti.at[i]], sem, add=True)
        for i in range(NCH)
    ]
    for d in descs:
        d.wait()
    plsc.subcore_barrier()
    pltpu.sync_copy(accum.at[pl.ds(sid * RPT, RPT)],
                    out_hbm.at[cid, pl.ds(sid * RPT, RPT), pl.ds(0, 16)])
    pltpu.sync_copy(accum.at[pl.ds(sid * RPT, RPT)],
                    out_hbm.at[cid, pl.ds(sid * RPT, RPT), pl.ds(16, 16)])


_deg_call = pl.kernel(
    _deg_body,
    out_type=jax.ShapeDtypeStruct((NC, NPAD, 32), jnp.float32),
    mesh=_mesh,
    compiler_params=_sc_params,
    scratch_types=[
        pltpu.VMEM((NCH, CHUNK), jnp.int32),
        pltpu.VMEM((CHUNK, 16), jnp.float32),
        pltpu.VMEM_SHARED((NPAD, 16), jnp.float32),
        pltpu.SemaphoreType.DMA,
    ],
)


def _scatter_body(g_hbm, src_hbm, dst_hbm, zrows_hbm, out_hbm,
                  srci, dsti, rows, accum, sg, ssc, si):
    cid = lax.axis_index("c")
    sid = lax.axis_index("s")
    wid = sid * NC + cid
    pltpu.sync_copy(zrows_hbm, accum.at[pl.ds(sid * RPT, RPT)])
    idx_descs = [pltpu.async_copy(src_hbm.at[pl.ds(wid * TPW, TPW)], srci, si)]
    idx_descs += [
        pltpu.async_copy(dst_hbm.at[pl.ds(wid * TPW + i * CHUNK, CHUNK)],
                         dsti.at[i], si)
        for i in range(NCH)
    ]
    for d in idx_descs:
        d.wait()
    plsc.subcore_barrier()

    def start_gather(i):
        b = i % NBUF
        return pltpu.async_copy(
            g_hbm.at[srci.at[pl.ds(i * CHUNK, CHUNK)]], rows[b], sg[b])

    gd = [None] * NCH
    sd = [None] * NCH
    for j in range(NBUF - 1):
        gd[j] = start_gather(j)
    last_waited = -1
    for i in range(NCH):
        b = i % NBUF
        gd[i].wait()
        sd[i] = pltpu.async_copy(rows[b], accum.at[dsti.at[i]], ssc[b],
                                 add=True)
        j = i + NBUF - 1
        if j < NCH:
            if i >= 1:
                sd[i - 1].wait()
                last_waited = i - 1
            gd[j] = start_gather(j)
    for k in range(last_waited + 1, NCH):
        sd[k].wait()
    plsc.subcore_barrier()
    pltpu.sync_copy(accum.at[pl.ds(sid * RPT, RPT)],
                    out_hbm.at[cid, pl.ds(sid * RPT, RPT)])


_scatter32 = pl.kernel(
    _scatter_body,
    out_type=jax.ShapeDtypeStruct((NC, NPAD, 32), jnp.float32),
    mesh=_mesh,
    compiler_params=_sc_params,
    scratch_types=[
        pltpu.VMEM((TPW,), jnp.int32),
        pltpu.VMEM((NCH, CHUNK), jnp.int32),
        [pltpu.VMEM((CHUNK, 32), jnp.float32) for _ in range(NBUF)],
        pltpu.VMEM_SHARED((NPAD, 32), jnp.float32),
        [pltpu.SemaphoreType.DMA for _ in range(NBUF)],
        [pltpu.SemaphoreType.DMA for _ in range(NBUF)],
        pltpu.SemaphoreType.DMA,
    ],
)


def _split_body(ei_ref, src_ref, dst_ref):
    src_ref[...] = ei_ref[0, :]
    dst_ref[...] = ei_ref[1, :]


_split = pl.pallas_call(
    _split_body,
    out_shape=(jax.ShapeDtypeStruct((E,), jnp.int32),
               jax.ShapeDtypeStruct((E,), jnp.int32)))


def _dinv_from(degp_ref):
    deg = 1.0 + degp_ref[0, 0:NF, :] + degp_ref[1, 0:NF, :]
    return lax.rsqrt(deg)


def _prep_body(degp_ref, xf_ref, w_ref, g_ref):
    dinv = _dinv_from(degp_ref)
    h = jnp.dot(xf_ref[...], w_ref[...], preferred_element_type=jnp.float32)
    g_ref[...] = h * dinv


_prep = pl.pallas_call(
    _prep_body, out_shape=jax.ShapeDtypeStruct((NF, 128), jnp.float32))


def _combine_body(degp_ref, p_ref, g_ref, w_ref, b_ref, o_ref):
    dinv = _dinv_from(degp_ref)
    s = p_ref[0, 0:NF, :] + p_ref[1, 0:NF, :] + g_ref[...]
    pre = s * dinv + b_ref[...]
    a = jnp.where(pre > 0, pre, jnp.exp(jnp.minimum(pre, 0.0)) - 1.0)
    o_ref[...] = jnp.dot(a, w_ref[...],
                         preferred_element_type=jnp.float32) * dinv


_combine = pl.pallas_call(
    _combine_body, out_shape=jax.ShapeDtypeStruct((NF, 128), jnp.float32))


def _final_body(degp_ref, p_ref, g_ref, b_ref, o_ref):
    dinv = _dinv_from(degp_ref)
    o_ref[...] = ((p_ref[0, 0:NF, :] + p_ref[1, 0:NF, :] + g_ref[...])
                  * dinv + b_ref[...])


_final = pl.pallas_call(
    _final_body, out_shape=jax.ShapeDtypeStruct((NF, 128), jnp.float32))


def _blockdiag4(w32):
    z = jnp.zeros((32, 32), jnp.float32)
    cols = [jnp.concatenate([z] * k + [w32] + [z] * (3 - k), axis=0)
            for k in range(4)]
    return jnp.concatenate(cols, axis=1)


def _pad2d(w, rows, cols):
    r, c = w.shape
    w = jnp.concatenate([w, jnp.zeros((rows - r, c), jnp.float32)], axis=0)
    return jnp.concatenate([w, jnp.zeros((rows, cols - c), jnp.float32)],
                           axis=1)


def kernel(x, edge_index, W1, b1, W2, b2, W3, b3):
    ei = edge_index.astype(jnp.int32)
    src, dst = _split(ei)
    f32 = jnp.float32

    xf = x.reshape(NF, 512)
    W1p = _pad2d(W1, 128, 32)
    z1 = jnp.zeros((128, 32), f32)
    W1blk = jnp.concatenate(
        [jnp.concatenate([z1] * k + [W1p] + [z1] * (3 - k), axis=0)
         for k in range(4)], axis=1).reshape(512, 128)
    W2blk = _blockdiag4(_pad2d(W2, 32, 32))
    W3blk = _blockdiag4(_pad2d(W3, 32, 32))

    def tile_bias(b):
        bp = jnp.concatenate([b, jnp.zeros((32 - b.shape[0],), f32)])
        return jnp.tile(bp, 4).reshape(1, 128)

    b1t, b2t, b3t = tile_bias(b1), tile_bias(b2), tile_bias(b3)
    ones16 = jnp.ones((CHUNK, 16), f32)
    z16 = jnp.zeros((RPT, 16), f32)
    z32 = jnp.zeros((RPT, 32), f32)

    def fold(p):
        return p.reshape(NC, NPAD // 4, 128)

    degp = fold(_deg_call(dst, ones16, z16))
    g1 = _prep(degp, xf, W1blk)
    p1 = fold(_scatter32(g1.reshape(N, 32), src, dst, z32))
    g2 = _combine(degp, p1, g1, W2blk, b1t)
    p2 = fold(_scatter32(g2.reshape(N, 32), src, dst, z32))
    g3 = _combine(degp, p2, g2, W3blk, b2t)
    p3 = fold(_scatter32(g3.reshape(N, 32), src, dst, z32))
    out = _final(degp, p3, g3, b3t)
    return out.reshape(N, 32)[:, :10]
